# baseline (speedup 1.0000x reference)
import functools
import math

import jax
import jax.numpy as jnp
from jax.experimental import pallas as pl
from jax.experimental.pallas import tpu as pltpu


def _cnt_body(dst_ref, et_ref, cnt_ref, *, ce, r):
    @pl.when(pl.program_id(0) == 0)
    def _():
        cnt_ref[...] = jnp.zeros_like(cnt_ref)

    iota_r = jax.lax.broadcasted_iota(jnp.int32, (1, r), 1)

    def body(i, carry):
        t = dst_ref[0, 0, i]
        rel = et_ref[0, 0, i]
        onehot = (iota_r == rel).astype(jnp.float32)
        cnt_ref[pl.ds(t, 1), :] += onehot
        return carry

    jax.lax.fori_loop(0, ce, body, 0)


def _acc_body(src_ref, dst_ref, et_ref, comp_ref, x_ref, cnt_ref, acc_ref,
              *, ce, r, nb):
    @pl.when(pl.program_id(0) == 0)
    def _():
        acc_ref[...] = jnp.zeros_like(acc_ref)

    iota_r = jax.lax.broadcasted_iota(jnp.int32, (1, r), 1)

    def body(i, carry):
        s = src_ref[0, 0, i]
        t = dst_ref[0, 0, i]
        rel = et_ref[0, 0, i]
        cnt_row = cnt_ref[pl.ds(t, 1), :]
        onehot = (iota_r == rel).astype(jnp.float32)
        winv = jnp.sum(cnt_row * onehot, axis=1, keepdims=True)
        w = 1.0 / jnp.maximum(winv, 1.0)
        xrow = x_ref[pl.ds(s, 1), :]
        for b in range(nb):
            cb = comp_ref[rel, b] * w
            acc_ref[b, pl.ds(t, 1), :] += cb * xrow
        return carry

    jax.lax.fori_loop(0, ce, body, 0)


def _dense_body(h_ref, acc_ref, bases_ref, root_ref, g_ref, b_ref, rs_ref,
                out_ref, *, nb, mode):
    h = h_ref[...]
    o = jnp.dot(h, root_ref[...], preferred_element_type=jnp.float32)
    for b in range(nb):
        o += jnp.dot(acc_ref[b], bases_ref[b],
                     preferred_element_type=jnp.float32)
    mu = jnp.mean(o, axis=-1, keepdims=True)
    var = jnp.mean((o - mu) ** 2, axis=-1, keepdims=True)
    ln = (o - mu) / jnp.sqrt(var + 1e-5) * g_ref[...] + b_ref[...]
    if mode == "relu":
        out_ref[...] = jnp.maximum(ln, 0.0)
    else:
        out_ref[...] = ln + rs_ref[0, 0] * h


def _chunk(e):
    for ce in (8000, 6400, 4000, 2000, 1000):
        if e % ce == 0:
            return ce
    return e


def _compute_cnt(dst2, et2, n, r, e):
    ce = _chunk(e)
    return pl.pallas_call(
        functools.partial(_cnt_body, ce=ce, r=r),
        grid=(e // ce,),
        in_specs=[
            pl.BlockSpec((1, 1, ce), lambda i: (i, 0, 0), memory_space=pltpu.SMEM),
            pl.BlockSpec((1, 1, ce), lambda i: (i, 0, 0), memory_space=pltpu.SMEM),
        ],
        out_specs=pl.BlockSpec((n, r), lambda i: (0, 0)),
        out_shape=jax.ShapeDtypeStruct((n, r), jnp.float32),
    )(dst2, et2)


def _compute_acc(src2, dst2, et2, comp, h, cnt, n, d, r, nb, e):
    ce = _chunk(e)
    return pl.pallas_call(
        functools.partial(_acc_body, ce=ce, r=r, nb=nb),
        grid=(e // ce,),
        in_specs=[
            pl.BlockSpec((1, 1, ce), lambda i: (i, 0, 0), memory_space=pltpu.SMEM),
            pl.BlockSpec((1, 1, ce), lambda i: (i, 0, 0), memory_space=pltpu.SMEM),
            pl.BlockSpec((1, 1, ce), lambda i: (i, 0, 0), memory_space=pltpu.SMEM),
            pl.BlockSpec((r, nb), lambda i: (0, 0), memory_space=pltpu.SMEM),
            pl.BlockSpec((n, d), lambda i: (0, 0)),
            pl.BlockSpec((n, r), lambda i: (0, 0)),
        ],
        out_specs=pl.BlockSpec((nb, n, d), lambda i: (0, 0, 0)),
        out_shape=jax.ShapeDtypeStruct((nb, n, d), jnp.float32),
    )(src2, dst2, et2, comp, h, cnt)


def _dense(h, acc, bases, root, gamma, beta, rs, n, d, nb, mode):
    bn = 400 if n % 400 == 0 else n
    return pl.pallas_call(
        functools.partial(_dense_body, nb=nb, mode=mode),
        grid=(n // bn,),
        in_specs=[
            pl.BlockSpec((bn, d), lambda i: (i, 0)),
            pl.BlockSpec((nb, bn, d), lambda i: (0, i, 0)),
            pl.BlockSpec((nb, d, d), lambda i: (0, 0, 0)),
            pl.BlockSpec((d, d), lambda i: (0, 0)),
            pl.BlockSpec((1, d), lambda i: (0, 0)),
            pl.BlockSpec((1, d), lambda i: (0, 0)),
            pl.BlockSpec((1, 1), lambda i: (0, 0), memory_space=pltpu.SMEM),
        ],
        out_specs=pl.BlockSpec((bn, d), lambda i: (i, 0)),
        out_shape=jax.ShapeDtypeStruct((n, d), jnp.float32),
    )(h, acc, bases, root, gamma, beta, rs)


def kernel(x, edge_index, edge_type, bases1, comp1, root1, gamma1, beta1,
           bases2, comp2, root2, gamma2, beta2, res_scale):
    n, d = x.shape
    e = edge_index.shape[1]
    r, nb = comp1.shape

    ce = _chunk(e)
    nch = e // ce
    src2 = edge_index[0].reshape(nch, 1, ce)
    dst2 = edge_index[1].reshape(nch, 1, ce)
    et2 = edge_type.reshape(nch, 1, ce)
    g1 = gamma1.reshape(1, d)
    b1 = beta1.reshape(1, d)
    g2 = gamma2.reshape(1, d)
    b2 = beta2.reshape(1, d)
    rs = res_scale.reshape(1, 1)

    cnt = _compute_cnt(dst2, et2, n, r, e)

    acc1 = _compute_acc(src2, dst2, et2, comp1, x, cnt, n, d, r, nb, e)
    h = _dense(x, acc1, bases1, root1, g1, b1, rs, n, d, nb, "relu")

    acc2 = _compute_acc(src2, dst2, et2, comp2, h, cnt, n, d, r, nb, e)
    out = _dense(h, acc2, bases2, root2, g2, b2, rs, n, d, nb, "res")
    return out
